# baseline (device time: 74715 ns/iter reference)
import jax
import jax.numpy as jnp
from jax import lax
from jax.experimental import pallas as pl
from jax.experimental.pallas import tpu as pltpu

N_DEV = 32
CUBE = 8
NCUBE = 4
NCHUNK = 8


def _logical_id(q, p):
    z = 2 * (q // 2) + p // 4
    j = 4 * (q % 2) + p % 4
    return 8 * z + j


def kernel(x, w_mat):
    m, _ = x.shape
    _, n = w_mat.shape
    r1 = m // CUBE
    r2 = r1 // NCUBE
    cw = n // NCHUNK

    def body(x_ref, w_ref, out_ref, acc_ref, red_ref, g1_ref, g2_ref,
             ss1, ss2, ss3, ss4, rs1, rs2, rs3, rs4):
        my = lax.axis_index("i")
        j = my % 8
        zplane = my // 8
        q = (j // 4) % 2 + 2 * (zplane // 2)
        p = j % 4 + 4 * (zplane % 2)
        my_row = p * r1 + q * r2

        cube_peers = [(p + o) % CUBE for o in range(1, CUBE)]
        group_peers = [(q + o) % NCUBE for o in range(1, NCUBE)]

        bar = pltpu.get_barrier_semaphore()
        for pp in cube_peers:
            pl.semaphore_signal(bar, inc=1, device_id=(_logical_id(q, pp),),
                                device_id_type=pl.DeviceIdType.MESH)
        for qq in group_peers:
            pl.semaphore_signal(bar, inc=1, device_id=(_logical_id(qq, p),),
                                device_id_type=pl.DeviceIdType.MESH)
        pl.semaphore_wait(bar, CUBE - 1 + NCUBE - 1)

        def col(c):
            return pl.ds(c * cw, cw)

        l1 = [[None] * (CUBE - 1) for _ in range(NCHUNK)]
        l2 = [[None] * (NCUBE - 1) for _ in range(NCHUNK)]
        l2b = [[None] * (NCUBE - 1) for _ in range(NCHUNK)]
        l1b = [[None] * (CUBE - 1) for _ in range(NCHUNK)]

        def stage_gemm_l1(c):
            acc_ref[:, col(c)] = jnp.dot(
                x_ref[...], w_ref[:, col(c)],
                preferred_element_type=jnp.float32,
            )
            for oi, pp in enumerate(cube_peers):
                rdma = pltpu.make_async_remote_copy(
                    src_ref=acc_ref.at[pl.ds(pp * r1, r1), col(c)],
                    dst_ref=g1_ref.at[oi, :, col(c)],
                    send_sem=ss1.at[c, oi],
                    recv_sem=rs1.at[c, oi],
                    device_id=(_logical_id(q, pp),),
                    device_id_type=pl.DeviceIdType.MESH,
                )
                rdma.start()
                l1[c][oi] = rdma

        def stage_l2(c):
            for rdma in l1[c]:
                rdma.wait_recv()
            red_ref[:, col(c)] = acc_ref[pl.ds(p * r1, r1), col(c)] + jnp.sum(
                g1_ref[:, :, col(c)], axis=0
            )
            for oi, qq in enumerate(group_peers):
                rdma = pltpu.make_async_remote_copy(
                    src_ref=red_ref.at[pl.ds(qq * r2, r2), col(c)],
                    dst_ref=g2_ref.at[oi, :, col(c)],
                    send_sem=ss2.at[c, oi],
                    recv_sem=rs2.at[c, oi],
                    device_id=(_logical_id(qq, p),),
                    device_id_type=pl.DeviceIdType.MESH,
                )
                rdma.start()
                l2[c][oi] = rdma

        def stage_l2b(c):
            for rdma in l2[c]:
                rdma.wait_recv()
            final = red_ref[pl.ds(q * r2, r2), col(c)] + jnp.sum(
                g2_ref[:, :, col(c)], axis=0
            )
            out_ref[pl.ds(my_row, r2), col(c)] = jnp.maximum(final, 0.0)
            for oi, qq in enumerate(group_peers):
                rdma = pltpu.make_async_remote_copy(
                    src_ref=out_ref.at[pl.ds(my_row, r2), col(c)],
                    dst_ref=out_ref.at[pl.ds(my_row, r2), col(c)],
                    send_sem=ss3.at[c, oi],
                    recv_sem=rs3.at[c, oi],
                    device_id=(_logical_id(qq, p),),
                    device_id_type=pl.DeviceIdType.MESH,
                )
                rdma.start()
                l2b[c][oi] = rdma

        def stage_l1b(c):
            for rdma in l2b[c]:
                rdma.wait_recv()
            for oi, pp in enumerate(cube_peers):
                rdma = pltpu.make_async_remote_copy(
                    src_ref=out_ref.at[pl.ds(p * r1, r1), col(c)],
                    dst_ref=out_ref.at[pl.ds(p * r1, r1), col(c)],
                    send_sem=ss4.at[c, oi],
                    recv_sem=rs4.at[c, oi],
                    device_id=(_logical_id(q, pp),),
                    device_id_type=pl.DeviceIdType.MESH,
                )
                rdma.start()
                l1b[c][oi] = rdma

        for t in range(NCHUNK + 3):
            if 0 <= t - 3 < NCHUNK:
                stage_l1b(t - 3)
            if 0 <= t - 2 < NCHUNK:
                stage_l2b(t - 2)
            if 0 <= t - 1 < NCHUNK:
                stage_l2(t - 1)
            if t < NCHUNK:
                stage_gemm_l1(t)

        for c in range(NCHUNK):
            for rdma in l1b[c]:
                rdma.wait_recv()
        for group in (l1, l2, l2b, l1b):
            for c in range(NCHUNK):
                for rdma in group[c]:
                    rdma.wait_send()

    return pl.pallas_call(
        body,
        out_shape=jax.ShapeDtypeStruct((m, n), jnp.float32),
        in_specs=[
            pl.BlockSpec(memory_space=pltpu.VMEM),
            pl.BlockSpec(memory_space=pltpu.VMEM),
        ],
        out_specs=pl.BlockSpec(memory_space=pltpu.VMEM),
        scratch_shapes=[
            pltpu.VMEM((m, n), jnp.float32),
            pltpu.VMEM((r1, n), jnp.float32),
            pltpu.VMEM((CUBE - 1, r1, n), jnp.float32),
            pltpu.VMEM((NCUBE - 1, r2, n), jnp.float32),
            pltpu.SemaphoreType.DMA((NCHUNK, CUBE - 1)),
            pltpu.SemaphoreType.DMA((NCHUNK, NCUBE - 1)),
            pltpu.SemaphoreType.DMA((NCHUNK, NCUBE - 1)),
            pltpu.SemaphoreType.DMA((NCHUNK, CUBE - 1)),
            pltpu.SemaphoreType.DMA((NCHUNK, CUBE - 1)),
            pltpu.SemaphoreType.DMA((NCHUNK, NCUBE - 1)),
            pltpu.SemaphoreType.DMA((NCHUNK, NCUBE - 1)),
            pltpu.SemaphoreType.DMA((NCHUNK, CUBE - 1)),
        ],
        compiler_params=pltpu.CompilerParams(collective_id=0),
    )(x, w_mat)


# device time: 69275 ns/iter; 1.0785x vs baseline; 1.0785x over previous
import jax
import jax.numpy as jnp
from jax import lax
from jax.experimental import pallas as pl
from jax.experimental.pallas import tpu as pltpu

N_DEV = 32
CUBE = 8
NCUBE = 4
NCHUNK = 8


def _logical_id(q, p):
    z = 2 * (q // 2) + p // 4
    j = 4 * (q % 2) + p % 4
    return 8 * z + j


def kernel(x, w_mat):
    m, _ = x.shape
    _, n = w_mat.shape
    r1 = m // CUBE
    r2 = r1 // NCUBE
    cw = n // NCHUNK

    def body(x_ref, w_ref, out_ref, acc_ref, red_ref, g1_ref, g2_ref,
             ss1, ss2, ss3, ss4, rs1, rs2, rs3, rs4):
        my = lax.axis_index("i")
        j = my % 8
        zplane = my // 8
        q = (j // 4) % 2 + 2 * (zplane // 2)
        p = j % 4 + 4 * (zplane % 2)
        my_row = p * r1 + q * r2

        cube_peers = [(p + o) % CUBE for o in range(1, CUBE)]
        group_peers = [(q + o) % NCUBE for o in range(1, NCUBE)]

        bar = pltpu.get_barrier_semaphore()
        for pp in cube_peers:
            pl.semaphore_signal(bar, inc=1, device_id=(_logical_id(q, pp),),
                                device_id_type=pl.DeviceIdType.MESH)
        for qq in group_peers:
            pl.semaphore_signal(bar, inc=1, device_id=(_logical_id(qq, p),),
                                device_id_type=pl.DeviceIdType.MESH)
        pl.semaphore_wait(bar, CUBE - 1 + NCUBE - 1)

        acc_ref[...] = jnp.dot(
            x_ref[...], w_ref[...], preferred_element_type=jnp.float32
        )

        def col(c):
            return pl.ds(c * cw, cw)

        l1 = [[None] * (CUBE - 1) for _ in range(NCHUNK)]
        for c in range(NCHUNK):
            for oi, pp in enumerate(cube_peers):
                rdma = pltpu.make_async_remote_copy(
                    src_ref=acc_ref.at[pl.ds(pp * r1, r1), col(c)],
                    dst_ref=g1_ref.at[oi, :, col(c)],
                    send_sem=ss1.at[c, oi],
                    recv_sem=rs1.at[c, oi],
                    device_id=(_logical_id(q, pp),),
                    device_id_type=pl.DeviceIdType.MESH,
                )
                rdma.start()
                l1[c][oi] = rdma

        l2 = [[None] * (NCUBE - 1) for _ in range(NCHUNK)]
        for c in range(NCHUNK):
            for rdma in l1[c]:
                rdma.wait_recv()
            red_ref[:, col(c)] = acc_ref[pl.ds(p * r1, r1), col(c)] + jnp.sum(
                g1_ref[:, :, col(c)], axis=0
            )
            for oi, qq in enumerate(group_peers):
                rdma = pltpu.make_async_remote_copy(
                    src_ref=red_ref.at[pl.ds(qq * r2, r2), col(c)],
                    dst_ref=g2_ref.at[oi, :, col(c)],
                    send_sem=ss2.at[c, oi],
                    recv_sem=rs2.at[c, oi],
                    device_id=(_logical_id(qq, p),),
                    device_id_type=pl.DeviceIdType.MESH,
                )
                rdma.start()
                l2[c][oi] = rdma

        l2b = [[None] * (NCUBE - 1) for _ in range(NCHUNK)]
        for c in range(NCHUNK):
            for rdma in l2[c]:
                rdma.wait_recv()
            final = red_ref[pl.ds(q * r2, r2), col(c)] + jnp.sum(
                g2_ref[:, :, col(c)], axis=0
            )
            out_ref[pl.ds(my_row, r2), col(c)] = jnp.maximum(final, 0.0)
            for oi, qq in enumerate(group_peers):
                rdma = pltpu.make_async_remote_copy(
                    src_ref=out_ref.at[pl.ds(my_row, r2), col(c)],
                    dst_ref=out_ref.at[pl.ds(my_row, r2), col(c)],
                    send_sem=ss3.at[c, oi],
                    recv_sem=rs3.at[c, oi],
                    device_id=(_logical_id(qq, p),),
                    device_id_type=pl.DeviceIdType.MESH,
                )
                rdma.start()
                l2b[c][oi] = rdma

        l1b = [[None] * (CUBE - 1) for _ in range(NCHUNK)]
        for c in range(NCHUNK):
            for rdma in l2b[c]:
                rdma.wait_recv()
            for oi, pp in enumerate(cube_peers):
                rdma = pltpu.make_async_remote_copy(
                    src_ref=out_ref.at[pl.ds(p * r1, r1), col(c)],
                    dst_ref=out_ref.at[pl.ds(p * r1, r1), col(c)],
                    send_sem=ss4.at[c, oi],
                    recv_sem=rs4.at[c, oi],
                    device_id=(_logical_id(q, pp),),
                    device_id_type=pl.DeviceIdType.MESH,
                )
                rdma.start()
                l1b[c][oi] = rdma

        for c in range(NCHUNK):
            for rdma in l1b[c]:
                rdma.wait_recv()
        for group in (l1, l2, l2b, l1b):
            for c in range(NCHUNK):
                for rdma in group[c]:
                    rdma.wait_send()

    return pl.pallas_call(
        body,
        out_shape=jax.ShapeDtypeStruct((m, n), jnp.float32),
        in_specs=[
            pl.BlockSpec(memory_space=pltpu.VMEM),
            pl.BlockSpec(memory_space=pltpu.VMEM),
        ],
        out_specs=pl.BlockSpec(memory_space=pltpu.VMEM),
        scratch_shapes=[
            pltpu.VMEM((m, n), jnp.float32),
            pltpu.VMEM((r1, n), jnp.float32),
            pltpu.VMEM((CUBE - 1, r1, n), jnp.float32),
            pltpu.VMEM((NCUBE - 1, r2, n), jnp.float32),
            pltpu.SemaphoreType.DMA((NCHUNK, CUBE - 1)),
            pltpu.SemaphoreType.DMA((NCHUNK, NCUBE - 1)),
            pltpu.SemaphoreType.DMA((NCHUNK, NCUBE - 1)),
            pltpu.SemaphoreType.DMA((NCHUNK, CUBE - 1)),
            pltpu.SemaphoreType.DMA((NCHUNK, CUBE - 1)),
            pltpu.SemaphoreType.DMA((NCHUNK, NCUBE - 1)),
            pltpu.SemaphoreType.DMA((NCHUNK, NCUBE - 1)),
            pltpu.SemaphoreType.DMA((NCHUNK, CUBE - 1)),
        ],
        compiler_params=pltpu.CompilerParams(collective_id=0),
    )(x, w_mat)
